# baseline (speedup 1.0000x reference)
import functools

import jax
import jax.numpy as jnp
from jax import lax
from jax.experimental import pallas as pl
from jax.experimental.pallas import tpu as pltpu
from jax.experimental.pallas import tpu_sc as plsc

N_FIELDS = 26
VOCAB = 100000
EMB_DIM = 16
N_CONT = 13
B = 16384
EMB_COLS = N_FIELDS * EMB_DIM
EPS = 1e-5

CH = 128
BLK = 512
H1P = 1024
H2P = 512
CONTP = 16
OUTP = 128


CHB = 8192
VH = 50048
VH2 = VOCAB - VH


def _sc_gather(tableT, xcatT):
    info = plsc.get_sparse_core_info()
    nc, ns = info.num_cores, info.num_subcores
    nw = nc * ns
    npairs = N_FIELDS * EMB_DIM
    per_w = npairs // nw
    assert npairs == per_w * nw
    nchunk = B // CHB
    mesh = plsc.VectorSubcoreMesh(core_axis_name="c", subcore_axis_name="s")

    @functools.partial(
        pl.kernel,
        mesh=mesh,
        out_type=jax.ShapeDtypeStruct((npairs, B), jnp.float32),
        scratch_types=[
            pltpu.VMEM((VH,), jnp.float32),
            pltpu.VMEM((VH2,), jnp.float32),
            pltpu.VMEM((CHB,), jnp.int32),
            pltpu.VMEM((CHB,), jnp.float32),
            pltpu.VMEM((CHB,), jnp.float32),
            pltpu.SemaphoreType.DMA,
            pltpu.SemaphoreType.DMA,
            pltpu.SemaphoreType.DMA,
            pltpu.SemaphoreType.DMA,
        ],
        compiler_params=pltpu.CompilerParams(needs_layout_passes=False),
    )
    def gk(table_hbm, idx_hbm, out_hbm, rA, rB, idx_v, o_v0, o_v1,
           semA, semB, s0, s1):
        wid = lax.axis_index("s") * nc + lax.axis_index("c")
        o_bufs = (o_v0, o_v1)
        o_sems = (s0, s1)

        def fd(p):
            pid = wid * per_w + p
            return pid, pid // EMB_DIM, pid % EMB_DIM

        pid0, f0, d0 = fd(0)
        pltpu.async_copy(table_hbm.at[f0, d0].at[pl.ds(0, VH)], rA, semA)
        pltpu.async_copy(table_hbm.at[f0, d0].at[pl.ds(VH, VH2)], rB, semB)

        def pair_body(p, carry):
            pid, f, d = fd(p)
            pltpu.make_async_copy(
                table_hbm.at[f, d].at[pl.ds(0, VH)], rA, semA).wait()

            for cb in range(nchunk):
                o_v = o_bufs[cb % 2]
                pltpu.sync_copy(idx_hbm.at[f, pl.ds(cb * CHB, CHB)], idx_v)

                @pl.when(p > 0)
                def _():
                    pltpu.make_async_copy(
                        o_v, out_hbm.at[pid, pl.ds(cb * CHB, CHB)],
                        o_sems[cb % 2]).wait()

                @plsc.parallel_loop(0, CHB // 16, unroll=8)
                def _(j):
                    o = j * 16
                    idx = idx_v[pl.ds(o, 16)]
                    ia = jnp.minimum(idx, VH - 1)
                    o_v[pl.ds(o, 16)] = plsc.load_gather(rA, [ia])

            pltpu.make_async_copy(
                table_hbm.at[f, d].at[pl.ds(VH, VH2)], rB, semB).wait()
            @pl.when(p + 1 < per_w)
            def _():
                pidn, fn, dn = fd(p + 1)
                pltpu.async_copy(table_hbm.at[fn, dn].at[pl.ds(0, VH)], rA, semA)

            for cb in range(nchunk):
                o_v = o_bufs[cb % 2]
                pltpu.sync_copy(idx_hbm.at[f, pl.ds(cb * CHB, CHB)], idx_v)

                @plsc.parallel_loop(0, CHB // 16, unroll=8)
                def _(j):
                    o = j * 16
                    idx = idx_v[pl.ds(o, 16)]
                    ib = jnp.maximum(idx - VH, 0)
                    g = plsc.load_gather(rB, [ib])
                    o_v[pl.ds(o, 16)] = jnp.where(idx >= VH, g, o_v[pl.ds(o, 16)])

                pltpu.async_copy(
                    o_v, out_hbm.at[pid, pl.ds(cb * CHB, CHB)],
                    o_sems[cb % 2])

            @pl.when(p + 1 < per_w)
            def _():
                pidn, fn, dn = fd(p + 1)
                pltpu.async_copy(table_hbm.at[fn, dn].at[pl.ds(VH, VH2)], rB, semB)
            return carry

        lax.fori_loop(0, per_w, pair_body, 0)
        pltpu.make_async_copy(o_v0, out_hbm.at[0, pl.ds(0, CHB)], s0).wait()
        pltpu.make_async_copy(o_v1, out_hbm.at[0, pl.ds(CHB, CHB)], s1).wait()

    return gk(tableT, xcatT)


NBLK = B // BLK


def _tc_mlp(xembT, xc, w1e, w1c, b1, bncg, bncb, w2, b2, g1, bb1,
            w3, b3, g2, bb2):

    def body(xt_ref, xc_ref, w1e_ref, w1c_ref, b1_ref, bncg_ref, bncb_ref,
             w2_ref, b2_ref, g1_ref, bb1_ref, w3_ref, b3_ref, g2_ref,
             bb2_ref, o_ref, h1_s, h2_s, st1, st2, affc, w2s, c2, w3s, c3):
        p = pl.program_id(0)
        i = pl.program_id(1)

        @pl.when((p == 0) & (i == 0))
        def _():
            xcf = xc_ref[...].astype(jnp.float32)
            m = jnp.mean(xcf, axis=0, keepdims=True)
            v = jnp.mean(xcf * xcf, axis=0, keepdims=True) - m * m
            sc = bncg_ref[...] / jnp.sqrt(v + EPS)
            affc[0:1, :] = sc
            affc[1:2, :] = bncb_ref[...] - m * sc
            st1[...] = jnp.zeros_like(st1)
            st2[...] = jnp.zeros_like(st2)
        ones = jnp.ones((1, BLK), jnp.float32)
        cdot = lambda a, b: lax.dot_general(
            a, b, (((0,), (0,)), ((), ())),
            preferred_element_type=jnp.float32)

        @pl.when(p == 0)
        def _():
            xcb = xc_ref[pl.ds(i * BLK, BLK), :].astype(jnp.float32)
            xcn = xcb * affc[0:1, :] + affc[1:2, :]
            h = lax.dot_general(xt_ref[...], w1e_ref[...],
                                (((0,), (0,)), ((), ())),
                                preferred_element_type=jnp.float32)
            h = h + jnp.dot(xcn, w1c_ref[...],
                            preferred_element_type=jnp.float32)
            h = jnp.maximum(h + b1_ref[...], 0.0)
            h1_s[pl.ds(i * BLK, BLK), :] = h.astype(jnp.bfloat16)
            st1[0:1, :] += jnp.dot(ones, h, preferred_element_type=jnp.float32)
            st1[1:2, :] += jnp.dot(ones, h * h, preferred_element_type=jnp.float32)

        @pl.when((p == 1) & (i == 0))
        def _():
            m = st1[0:1, :] * (1.0 / B)
            v = st1[1:2, :] * (1.0 / B) - m * m
            sc = g1_ref[...] / jnp.sqrt(v + EPS)
            sh = bb1_ref[...] - m * sc
            bT = jnp.swapaxes(jnp.concatenate([sc, sh], axis=0), 0, 1)
            w2f = w2_ref[...].astype(jnp.float32)
            w2s[...] = (w2f * bT[:, 0:1]).astype(jnp.bfloat16)
            c2[0:1, :] = b2_ref[...] + cdot(bT[:, 1:2], w2f)

        @pl.when(p == 1)
        def _():
            h1b = h1_s[pl.ds(i * BLK, BLK), :]
            h = jnp.dot(h1b, w2s[...], preferred_element_type=jnp.float32)
            h = jnp.maximum(h + c2[0:1, :], 0.0)
            h2_s[pl.ds(i * BLK, BLK), :] = h.astype(jnp.bfloat16)
            st2[0:1, :] += jnp.dot(ones, h, preferred_element_type=jnp.float32)
            st2[1:2, :] += jnp.dot(ones, h * h, preferred_element_type=jnp.float32)

        @pl.when((p == 2) & (i == 0))
        def _():
            m = st2[0:1, :] * (1.0 / B)
            v = st2[1:2, :] * (1.0 / B) - m * m
            sc = g2_ref[...] / jnp.sqrt(v + EPS)
            sh = bb2_ref[...] - m * sc
            bT = jnp.swapaxes(jnp.concatenate([sc, sh], axis=0), 0, 1)
            w3f = w3_ref[...].astype(jnp.float32)
            w3s[...] = (w3f * bT[:, 0:1]).astype(jnp.bfloat16)
            c3[0:1, :] = b3_ref[...] + cdot(bT[:, 1:2], w3f)

        @pl.when(p == 2)
        def _():
            h2b = h2_s[pl.ds(i * BLK, BLK), :]
            o = jnp.dot(h2b, w3s[...], preferred_element_type=jnp.float32)
            o_ref[...] = o + c3[0:1, :]

    cnst = lambda p, i: (0, 0)
    return pl.pallas_call(
        body,
        grid=(3, NBLK),
        in_specs=[
            pl.BlockSpec((EMB_COLS, BLK),
                         lambda p, i: (0, jnp.where(p == 0, i, NBLK - 1))),
            pl.BlockSpec((B, CONTP), cnst),
            pl.BlockSpec((EMB_COLS, H1P), cnst),
            pl.BlockSpec((CONTP, H1P), cnst),
            pl.BlockSpec((1, H1P), cnst),
            pl.BlockSpec((1, CONTP), cnst),
            pl.BlockSpec((1, CONTP), cnst),
            pl.BlockSpec((H1P, H2P), cnst),
            pl.BlockSpec((1, H2P), cnst),
            pl.BlockSpec((1, H1P), cnst),
            pl.BlockSpec((1, H1P), cnst),
            pl.BlockSpec((H2P, OUTP), cnst),
            pl.BlockSpec((1, OUTP), cnst),
            pl.BlockSpec((1, H2P), cnst),
            pl.BlockSpec((1, H2P), cnst),
        ],
        out_specs=pl.BlockSpec((BLK, OUTP),
                               lambda p, i: (jnp.where(p == 2, i, 0), 0)),
        out_shape=jax.ShapeDtypeStruct((B, OUTP), jnp.float32),
        scratch_shapes=[
            pltpu.VMEM((B, H1P), jnp.bfloat16),
            pltpu.VMEM((B, H2P), jnp.bfloat16),
            pltpu.VMEM((8, H1P), jnp.float32),
            pltpu.VMEM((8, H2P), jnp.float32),
            pltpu.VMEM((8, CONTP), jnp.float32),
            pltpu.VMEM((H1P, H2P), jnp.bfloat16),
            pltpu.VMEM((8, H2P), jnp.float32),
            pltpu.VMEM((H2P, OUTP), jnp.bfloat16),
            pltpu.VMEM((8, OUTP), jnp.float32),
        ],
        compiler_params=pltpu.CompilerParams(
            dimension_semantics=("arbitrary", "arbitrary"),
            vmem_limit_bytes=64 * 1024 * 1024),
    )(xembT, xc, w1e, w1c, b1, bncg, bncb, w2, b2, g1, bb1, w3, b3, g2, bb2)


def kernel(x_cat, x_cont, emb_tables, bn_cont_g, bn_cont_b,
           W1, b1, bn1_g, bn1_b, W2, b2, bn2_g, bn2_b, W3, b3):
    tableT = jnp.transpose(emb_tables, (0, 2, 1))
    xcatT = jnp.transpose(x_cat.astype(jnp.int32), (1, 0))

    xc = jnp.pad(x_cont, ((0, 0), (0, CONTP - N_CONT))).astype(jnp.bfloat16)
    bncg = jnp.pad(bn_cont_g, (0, CONTP - N_CONT)).reshape(1, CONTP)
    bncb = jnp.pad(bn_cont_b, (0, CONTP - N_CONT)).reshape(1, CONTP)

    n1 = W1.shape[0]
    n2 = W2.shape[0]
    w1e = jnp.pad(W1[:, :EMB_COLS].T, ((0, 0), (0, H1P - n1)))
    w1c = jnp.pad(W1[:, EMB_COLS:].T,
                  ((0, CONTP - N_CONT), (0, H1P - n1)))
    b1p = jnp.pad(b1, (0, H1P - n1)).reshape(1, H1P)
    g1p = jnp.pad(bn1_g, (0, H1P - n1)).reshape(1, H1P)
    bb1p = jnp.pad(bn1_b, (0, H1P - n1)).reshape(1, H1P)

    w2p = jnp.pad(W2.T, ((0, H1P - n1), (0, H2P - n2))).astype(jnp.bfloat16)
    b2p = jnp.pad(b2, (0, H2P - n2)).reshape(1, H2P)
    g2p = jnp.pad(bn2_g, (0, H2P - n2)).reshape(1, H2P)
    bb2p = jnp.pad(bn2_b, (0, H2P - n2)).reshape(1, H2P)

    w3p = jnp.pad(W3.T, ((0, H2P - n2), (0, OUTP - 1))).astype(jnp.bfloat16)
    b3p = jnp.pad(b3, (0, OUTP - 1)).reshape(1, OUTP)

    xembT = _sc_gather(tableT, xcatT)
    o = _tc_mlp(xembT, xc, w1e, w1c, b1p, bncg, bncb,
                w2p, b2p, g1p, bb1p, w3p, b3p, g2p, bb2p)
    return o[:, :1]


# speedup vs baseline: 1.1317x; 1.1317x over previous
import functools

import jax
import jax.numpy as jnp
from jax import lax
from jax.experimental import pallas as pl
from jax.experimental.pallas import tpu as pltpu
from jax.experimental.pallas import tpu_sc as plsc

N_FIELDS = 26
VOCAB = 100000
EMB_DIM = 16
N_CONT = 13
B = 16384
EMB_COLS = N_FIELDS * EMB_DIM
EPS = 1e-5

CH = 128
BLK = 512
H1P = 1024
H2P = 512
CONTP = 16
OUTP = 128


CHB = 8192


def _sc_gather(tableT, xcatT):
    info = plsc.get_sparse_core_info()
    nc, ns = info.num_cores, info.num_subcores
    nw = nc * ns
    npairs = N_FIELDS * EMB_DIM
    per_w = npairs // nw
    assert npairs == per_w * nw
    nchunk = B // CHB
    mesh = plsc.VectorSubcoreMesh(core_axis_name="c", subcore_axis_name="s")

    @functools.partial(
        pl.kernel,
        mesh=mesh,
        out_type=jax.ShapeDtypeStruct((npairs, B), jnp.float32),
        scratch_types=[
            pltpu.VMEM((VOCAB,), jnp.float32),
            pltpu.VMEM((CHB,), jnp.int32),
            pltpu.VMEM((CHB,), jnp.float32),
            pltpu.VMEM((CHB,), jnp.float32),
            pltpu.SemaphoreType.DMA,
            pltpu.SemaphoreType.DMA,
            pltpu.SemaphoreType.DMA,
        ],
        compiler_params=pltpu.CompilerParams(needs_layout_passes=False),
    )
    def gk(table_hbm, idx_hbm, out_hbm, row_v, idx_v, o_v0, o_v1,
           rsem, s0, s1):
        wid = lax.axis_index("s") * nc + lax.axis_index("c")
        o_bufs = (o_v0, o_v1)
        o_sems = (s0, s1)

        def pair_body(p, carry):
            pid = wid * per_w + p
            f = pid // EMB_DIM
            d = pid % EMB_DIM
            rcp = pltpu.async_copy(table_hbm.at[f, d], row_v, rsem)
            rcp.wait()

            for cb in range(nchunk):
                o_v = o_bufs[cb % 2]
                pltpu.sync_copy(idx_hbm.at[f, pl.ds(cb * CHB, CHB)], idx_v)

                @pl.when(p > 0)
                def _():
                    pltpu.make_async_copy(
                        o_v, out_hbm.at[pid, pl.ds(cb * CHB, CHB)],
                        o_sems[cb % 2]).wait()

                @plsc.parallel_loop(0, CHB // 16, unroll=16)
                def _(j):
                    o = j * 16
                    idx = idx_v[pl.ds(o, 16)]
                    o_v[pl.ds(o, 16)] = plsc.load_gather(row_v, [idx])

                pltpu.async_copy(
                    o_v, out_hbm.at[pid, pl.ds(cb * CHB, CHB)],
                    o_sems[cb % 2])
            return carry

        lax.fori_loop(0, per_w, pair_body, 0)
        pltpu.make_async_copy(o_v0, out_hbm.at[0, pl.ds(0, CHB)], s0).wait()
        pltpu.make_async_copy(o_v1, out_hbm.at[0, pl.ds(CHB, CHB)], s1).wait()

    return gk(tableT, xcatT)


NBLK = B // BLK


def _tc_mlp(xembT, xc, w1e, w1c, b1, bncg, bncb, w2, b2, g1, bb1,
            w3, b3, g2, bb2):

    def body(xt_ref, xc_ref, w1e_ref, w1c_ref, b1_ref, bncg_ref, bncb_ref,
             w2_ref, b2_ref, g1_ref, bb1_ref, w3_ref, b3_ref, g2_ref,
             bb2_ref, o_ref, h1_s, h2_s, st1, st2, affc, w2s, c2, w3s, c3):
        p = pl.program_id(0)
        i = pl.program_id(1)

        @pl.when((p == 0) & (i == 0))
        def _():
            xcf = xc_ref[...].astype(jnp.float32)
            m = jnp.mean(xcf, axis=0, keepdims=True)
            v = jnp.mean(xcf * xcf, axis=0, keepdims=True) - m * m
            sc = bncg_ref[...] / jnp.sqrt(v + EPS)
            affc[0:1, :] = sc
            affc[1:2, :] = bncb_ref[...] - m * sc
            st1[...] = jnp.zeros_like(st1)
            st2[...] = jnp.zeros_like(st2)
        ones = jnp.ones((1, BLK), jnp.float32)
        cdot = lambda a, b: lax.dot_general(
            a, b, (((0,), (0,)), ((), ())),
            preferred_element_type=jnp.float32)

        @pl.when(p == 0)
        def _():
            xcb = xc_ref[pl.ds(i * BLK, BLK), :].astype(jnp.float32)
            xcn = xcb * affc[0:1, :] + affc[1:2, :]
            h = lax.dot_general(xt_ref[...], w1e_ref[...],
                                (((0,), (0,)), ((), ())),
                                preferred_element_type=jnp.float32)
            h = h + jnp.dot(xcn, w1c_ref[...],
                            preferred_element_type=jnp.float32)
            h = jnp.maximum(h + b1_ref[...], 0.0)
            h1_s[pl.ds(i * BLK, BLK), :] = h.astype(jnp.bfloat16)
            st1[0:1, :] += jnp.dot(ones, h, preferred_element_type=jnp.float32)
            st1[1:2, :] += jnp.dot(ones, h * h, preferred_element_type=jnp.float32)

        @pl.when((p == 1) & (i == 0))
        def _():
            m = st1[0:1, :] * (1.0 / B)
            v = st1[1:2, :] * (1.0 / B) - m * m
            sc = g1_ref[...] / jnp.sqrt(v + EPS)
            sh = bb1_ref[...] - m * sc
            bT = jnp.swapaxes(jnp.concatenate([sc, sh], axis=0), 0, 1)
            w2f = w2_ref[...].astype(jnp.float32)
            w2s[...] = (w2f * bT[:, 0:1]).astype(jnp.bfloat16)
            c2[0:1, :] = b2_ref[...] + cdot(bT[:, 1:2], w2f)

        @pl.when(p == 1)
        def _():
            h1b = h1_s[pl.ds(i * BLK, BLK), :]
            h = jnp.dot(h1b, w2s[...], preferred_element_type=jnp.float32)
            h = jnp.maximum(h + c2[0:1, :], 0.0)
            h2_s[pl.ds(i * BLK, BLK), :] = h.astype(jnp.bfloat16)
            st2[0:1, :] += jnp.dot(ones, h, preferred_element_type=jnp.float32)
            st2[1:2, :] += jnp.dot(ones, h * h, preferred_element_type=jnp.float32)

        @pl.when((p == 2) & (i == 0))
        def _():
            m = st2[0:1, :] * (1.0 / B)
            v = st2[1:2, :] * (1.0 / B) - m * m
            sc = g2_ref[...] / jnp.sqrt(v + EPS)
            sh = bb2_ref[...] - m * sc
            bT = jnp.swapaxes(jnp.concatenate([sc, sh], axis=0), 0, 1)
            w3f = w3_ref[...].astype(jnp.float32)
            w3s[...] = (w3f * bT[:, 0:1]).astype(jnp.bfloat16)
            c3[0:1, :] = b3_ref[...] + cdot(bT[:, 1:2], w3f)

        @pl.when(p == 2)
        def _():
            h2b = h2_s[pl.ds(i * BLK, BLK), :]
            o = jnp.dot(h2b, w3s[...], preferred_element_type=jnp.float32)
            o_ref[...] = o + c3[0:1, :]

    cnst = lambda p, i: (0, 0)
    return pl.pallas_call(
        body,
        grid=(3, NBLK),
        in_specs=[
            pl.BlockSpec((EMB_COLS, BLK),
                         lambda p, i: (0, jnp.where(p == 0, i, NBLK - 1))),
            pl.BlockSpec((B, CONTP), cnst),
            pl.BlockSpec((EMB_COLS, H1P), cnst),
            pl.BlockSpec((CONTP, H1P), cnst),
            pl.BlockSpec((1, H1P), cnst),
            pl.BlockSpec((1, CONTP), cnst),
            pl.BlockSpec((1, CONTP), cnst),
            pl.BlockSpec((H1P, H2P), cnst),
            pl.BlockSpec((1, H2P), cnst),
            pl.BlockSpec((1, H1P), cnst),
            pl.BlockSpec((1, H1P), cnst),
            pl.BlockSpec((H2P, OUTP), cnst),
            pl.BlockSpec((1, OUTP), cnst),
            pl.BlockSpec((1, H2P), cnst),
            pl.BlockSpec((1, H2P), cnst),
        ],
        out_specs=pl.BlockSpec((BLK, OUTP),
                               lambda p, i: (jnp.where(p == 2, i, 0), 0)),
        out_shape=jax.ShapeDtypeStruct((B, OUTP), jnp.float32),
        scratch_shapes=[
            pltpu.VMEM((B, H1P), jnp.bfloat16),
            pltpu.VMEM((B, H2P), jnp.bfloat16),
            pltpu.VMEM((8, H1P), jnp.float32),
            pltpu.VMEM((8, H2P), jnp.float32),
            pltpu.VMEM((8, CONTP), jnp.float32),
            pltpu.VMEM((H1P, H2P), jnp.bfloat16),
            pltpu.VMEM((8, H2P), jnp.float32),
            pltpu.VMEM((H2P, OUTP), jnp.bfloat16),
            pltpu.VMEM((8, OUTP), jnp.float32),
        ],
        compiler_params=pltpu.CompilerParams(
            dimension_semantics=("arbitrary", "arbitrary"),
            vmem_limit_bytes=64 * 1024 * 1024),
    )(xembT, xc, w1e, w1c, b1, bncg, bncb, w2, b2, g1, bb1, w3, b3, g2, bb2)


def kernel(x_cat, x_cont, emb_tables, bn_cont_g, bn_cont_b,
           W1, b1, bn1_g, bn1_b, W2, b2, bn2_g, bn2_b, W3, b3):
    tableT = jnp.transpose(emb_tables, (0, 2, 1))
    xcatT = jnp.transpose(x_cat.astype(jnp.int32), (1, 0))

    xc = jnp.pad(x_cont, ((0, 0), (0, CONTP - N_CONT))).astype(jnp.bfloat16)
    bncg = jnp.pad(bn_cont_g, (0, CONTP - N_CONT)).reshape(1, CONTP)
    bncb = jnp.pad(bn_cont_b, (0, CONTP - N_CONT)).reshape(1, CONTP)

    n1 = W1.shape[0]
    n2 = W2.shape[0]
    w1e = jnp.pad(W1[:, :EMB_COLS].T, ((0, 0), (0, H1P - n1)))
    w1c = jnp.pad(W1[:, EMB_COLS:].T,
                  ((0, CONTP - N_CONT), (0, H1P - n1)))
    b1p = jnp.pad(b1, (0, H1P - n1)).reshape(1, H1P)
    g1p = jnp.pad(bn1_g, (0, H1P - n1)).reshape(1, H1P)
    bb1p = jnp.pad(bn1_b, (0, H1P - n1)).reshape(1, H1P)

    w2p = jnp.pad(W2.T, ((0, H1P - n1), (0, H2P - n2))).astype(jnp.bfloat16)
    b2p = jnp.pad(b2, (0, H2P - n2)).reshape(1, H2P)
    g2p = jnp.pad(bn2_g, (0, H2P - n2)).reshape(1, H2P)
    bb2p = jnp.pad(bn2_b, (0, H2P - n2)).reshape(1, H2P)

    w3p = jnp.pad(W3.T, ((0, H2P - n2), (0, OUTP - 1))).astype(jnp.bfloat16)
    b3p = jnp.pad(b3, (0, OUTP - 1)).reshape(1, OUTP)

    xembT = _sc_gather(tableT, xcatT)
    o = _tc_mlp(xembT, xc, w1e, w1c, b1p, bncg, bncb,
                w2p, b2p, g1p, bb1p, w3p, b3p, g2p, bb2p)
    return o[:, :1]


# speedup vs baseline: 1.1342x; 1.0022x over previous
import functools

import jax
import jax.numpy as jnp
from jax import lax
from jax.experimental import pallas as pl
from jax.experimental.pallas import tpu as pltpu
from jax.experimental.pallas import tpu_sc as plsc

N_FIELDS = 26
VOCAB = 100000
EMB_DIM = 16
N_CONT = 13
B = 16384
EMB_COLS = N_FIELDS * EMB_DIM
EPS = 1e-5

CH = 128
BLK = 512
H1P = 1024
H2P = 512
CONTP = 16
OUTP = 128


CHB = 8192


def _sc_gather(tableT, xcatT):
    info = plsc.get_sparse_core_info()
    nc, ns = info.num_cores, info.num_subcores
    nw = nc * ns
    npairs = N_FIELDS * EMB_DIM
    per_w = npairs // nw
    assert npairs == per_w * nw
    nchunk = B // CHB
    mesh = plsc.VectorSubcoreMesh(core_axis_name="c", subcore_axis_name="s")

    @functools.partial(
        pl.kernel,
        mesh=mesh,
        out_type=jax.ShapeDtypeStruct((npairs, B), jnp.float32),
        scratch_types=[
            pltpu.VMEM((VOCAB,), jnp.float32),
            pltpu.VMEM((CHB,), jnp.int32),
            pltpu.VMEM((CHB,), jnp.float32),
            pltpu.VMEM((CHB,), jnp.float32),
            pltpu.SemaphoreType.DMA,
            pltpu.SemaphoreType.DMA,
            pltpu.SemaphoreType.DMA,
        ],
        compiler_params=pltpu.CompilerParams(needs_layout_passes=False),
    )
    def gk(table_hbm, idx_hbm, out_hbm, row_v, idx_v, o_v0, o_v1,
           rsem, s0, s1):
        wid = lax.axis_index("s") * nc + lax.axis_index("c")
        o_bufs = (o_v0, o_v1)
        o_sems = (s0, s1)

        def pair_body(p, carry):
            pid = wid * per_w + p
            f = pid // EMB_DIM
            d = pid % EMB_DIM
            rcp = pltpu.async_copy(table_hbm.at[f, d], row_v, rsem)
            rcp.wait()

            for cb in range(nchunk):
                o_v = o_bufs[cb % 2]
                pltpu.sync_copy(idx_hbm.at[f, pl.ds(cb * CHB, CHB)], idx_v)

                @pl.when(p > 0)
                def _():
                    pltpu.make_async_copy(
                        o_v, out_hbm.at[pid, pl.ds(cb * CHB, CHB)],
                        o_sems[cb % 2]).wait()

                @plsc.parallel_loop(0, CHB // 256, unroll=1)
                def _(j):
                    base = j * 256
                    idxs = [idx_v[pl.ds(base + u * 16, 16)]
                            for u in range(16)]
                    vals = [plsc.load_gather(row_v, [ix]) for ix in idxs]
                    for u in range(16):
                        o_v[pl.ds(base + u * 16, 16)] = vals[u]

                pltpu.async_copy(
                    o_v, out_hbm.at[pid, pl.ds(cb * CHB, CHB)],
                    o_sems[cb % 2])
            return carry

        lax.fori_loop(0, per_w, pair_body, 0)
        pltpu.make_async_copy(o_v0, out_hbm.at[0, pl.ds(0, CHB)], s0).wait()
        pltpu.make_async_copy(o_v1, out_hbm.at[0, pl.ds(CHB, CHB)], s1).wait()

    return gk(tableT, xcatT)


NBLK = B // BLK


def _tc_mlp(xembT, xc, w1e, w1c, b1, bncg, bncb, w2, b2, g1, bb1,
            w3, b3, g2, bb2):

    def body(xt_ref, xc_ref, w1e_ref, w1c_ref, b1_ref, bncg_ref, bncb_ref,
             w2_ref, b2_ref, g1_ref, bb1_ref, w3_ref, b3_ref, g2_ref,
             bb2_ref, o_ref, h1_s, h2_s, st1, st2, affc, w2s, c2, w3s, c3):
        p = pl.program_id(0)
        i = pl.program_id(1)

        @pl.when((p == 0) & (i == 0))
        def _():
            xcf = xc_ref[...].astype(jnp.float32)
            m = jnp.mean(xcf, axis=0, keepdims=True)
            v = jnp.mean(xcf * xcf, axis=0, keepdims=True) - m * m
            sc = bncg_ref[...] / jnp.sqrt(v + EPS)
            affc[0:1, :] = sc
            affc[1:2, :] = bncb_ref[...] - m * sc
            st1[...] = jnp.zeros_like(st1)
            st2[...] = jnp.zeros_like(st2)
        ones = jnp.ones((1, BLK), jnp.float32)
        cdot = lambda a, b: lax.dot_general(
            a, b, (((0,), (0,)), ((), ())),
            preferred_element_type=jnp.float32)

        @pl.when(p == 0)
        def _():
            xcb = xc_ref[pl.ds(i * BLK, BLK), :].astype(jnp.float32)
            xcn = xcb * affc[0:1, :] + affc[1:2, :]
            h = lax.dot_general(xt_ref[...], w1e_ref[...],
                                (((0,), (0,)), ((), ())),
                                preferred_element_type=jnp.float32)
            h = h + jnp.dot(xcn, w1c_ref[...],
                            preferred_element_type=jnp.float32)
            h = jnp.maximum(h + b1_ref[...], 0.0)
            h1_s[pl.ds(i * BLK, BLK), :] = h.astype(jnp.bfloat16)
            st1[0:1, :] += jnp.dot(ones, h, preferred_element_type=jnp.float32)
            st1[1:2, :] += jnp.dot(ones, h * h, preferred_element_type=jnp.float32)

        @pl.when((p == 1) & (i == 0))
        def _():
            m = st1[0:1, :] * (1.0 / B)
            v = st1[1:2, :] * (1.0 / B) - m * m
            sc = g1_ref[...] / jnp.sqrt(v + EPS)
            sh = bb1_ref[...] - m * sc
            bT = jnp.swapaxes(jnp.concatenate([sc, sh], axis=0), 0, 1)
            w2f = w2_ref[...].astype(jnp.float32)
            w2s[...] = (w2f * bT[:, 0:1]).astype(jnp.bfloat16)
            c2[0:1, :] = b2_ref[...] + cdot(bT[:, 1:2], w2f)

        @pl.when(p == 1)
        def _():
            h1b = h1_s[pl.ds(i * BLK, BLK), :]
            h = jnp.dot(h1b, w2s[...], preferred_element_type=jnp.float32)
            h = jnp.maximum(h + c2[0:1, :], 0.0)
            h2_s[pl.ds(i * BLK, BLK), :] = h.astype(jnp.bfloat16)
            st2[0:1, :] += jnp.dot(ones, h, preferred_element_type=jnp.float32)
            st2[1:2, :] += jnp.dot(ones, h * h, preferred_element_type=jnp.float32)

        @pl.when((p == 2) & (i == 0))
        def _():
            m = st2[0:1, :] * (1.0 / B)
            v = st2[1:2, :] * (1.0 / B) - m * m
            sc = g2_ref[...] / jnp.sqrt(v + EPS)
            sh = bb2_ref[...] - m * sc
            bT = jnp.swapaxes(jnp.concatenate([sc, sh], axis=0), 0, 1)
            w3f = w3_ref[...].astype(jnp.float32)
            w3s[...] = (w3f * bT[:, 0:1]).astype(jnp.bfloat16)
            c3[0:1, :] = b3_ref[...] + cdot(bT[:, 1:2], w3f)

        @pl.when(p == 2)
        def _():
            h2b = h2_s[pl.ds(i * BLK, BLK), :]
            o = jnp.dot(h2b, w3s[...], preferred_element_type=jnp.float32)
            o_ref[...] = o + c3[0:1, :]

    cnst = lambda p, i: (0, 0)
    return pl.pallas_call(
        body,
        grid=(3, NBLK),
        in_specs=[
            pl.BlockSpec((EMB_COLS, BLK),
                         lambda p, i: (0, jnp.where(p == 0, i, NBLK - 1))),
            pl.BlockSpec((B, CONTP), cnst),
            pl.BlockSpec((EMB_COLS, H1P), cnst),
            pl.BlockSpec((CONTP, H1P), cnst),
            pl.BlockSpec((1, H1P), cnst),
            pl.BlockSpec((1, CONTP), cnst),
            pl.BlockSpec((1, CONTP), cnst),
            pl.BlockSpec((H1P, H2P), cnst),
            pl.BlockSpec((1, H2P), cnst),
            pl.BlockSpec((1, H1P), cnst),
            pl.BlockSpec((1, H1P), cnst),
            pl.BlockSpec((H2P, OUTP), cnst),
            pl.BlockSpec((1, OUTP), cnst),
            pl.BlockSpec((1, H2P), cnst),
            pl.BlockSpec((1, H2P), cnst),
        ],
        out_specs=pl.BlockSpec((BLK, OUTP),
                               lambda p, i: (jnp.where(p == 2, i, 0), 0)),
        out_shape=jax.ShapeDtypeStruct((B, OUTP), jnp.float32),
        scratch_shapes=[
            pltpu.VMEM((B, H1P), jnp.bfloat16),
            pltpu.VMEM((B, H2P), jnp.bfloat16),
            pltpu.VMEM((8, H1P), jnp.float32),
            pltpu.VMEM((8, H2P), jnp.float32),
            pltpu.VMEM((8, CONTP), jnp.float32),
            pltpu.VMEM((H1P, H2P), jnp.bfloat16),
            pltpu.VMEM((8, H2P), jnp.float32),
            pltpu.VMEM((H2P, OUTP), jnp.bfloat16),
            pltpu.VMEM((8, OUTP), jnp.float32),
        ],
        compiler_params=pltpu.CompilerParams(
            dimension_semantics=("arbitrary", "arbitrary"),
            vmem_limit_bytes=64 * 1024 * 1024),
    )(xembT, xc, w1e, w1c, b1, bncg, bncb, w2, b2, g1, bb1, w3, b3, g2, bb2)


def kernel(x_cat, x_cont, emb_tables, bn_cont_g, bn_cont_b,
           W1, b1, bn1_g, bn1_b, W2, b2, bn2_g, bn2_b, W3, b3):
    tableT = jnp.transpose(emb_tables, (0, 2, 1))
    xcatT = jnp.transpose(x_cat.astype(jnp.int32), (1, 0))

    xc = jnp.pad(x_cont, ((0, 0), (0, CONTP - N_CONT))).astype(jnp.bfloat16)
    bncg = jnp.pad(bn_cont_g, (0, CONTP - N_CONT)).reshape(1, CONTP)
    bncb = jnp.pad(bn_cont_b, (0, CONTP - N_CONT)).reshape(1, CONTP)

    n1 = W1.shape[0]
    n2 = W2.shape[0]
    w1e = jnp.pad(W1[:, :EMB_COLS].T, ((0, 0), (0, H1P - n1)))
    w1c = jnp.pad(W1[:, EMB_COLS:].T,
                  ((0, CONTP - N_CONT), (0, H1P - n1)))
    b1p = jnp.pad(b1, (0, H1P - n1)).reshape(1, H1P)
    g1p = jnp.pad(bn1_g, (0, H1P - n1)).reshape(1, H1P)
    bb1p = jnp.pad(bn1_b, (0, H1P - n1)).reshape(1, H1P)

    w2p = jnp.pad(W2.T, ((0, H1P - n1), (0, H2P - n2))).astype(jnp.bfloat16)
    b2p = jnp.pad(b2, (0, H2P - n2)).reshape(1, H2P)
    g2p = jnp.pad(bn2_g, (0, H2P - n2)).reshape(1, H2P)
    bb2p = jnp.pad(bn2_b, (0, H2P - n2)).reshape(1, H2P)

    w3p = jnp.pad(W3.T, ((0, H2P - n2), (0, OUTP - 1))).astype(jnp.bfloat16)
    b3p = jnp.pad(b3, (0, OUTP - 1)).reshape(1, OUTP)

    xembT = _sc_gather(tableT, xcatT)
    o = _tc_mlp(xembT, xc, w1e, w1c, b1p, bncg, bncb,
                w2p, b2p, g1p, bb1p, w3p, b3p, g2p, bb2p)
    return o[:, :1]


# speedup vs baseline: 1.2365x; 1.0902x over previous
import functools

import jax
import jax.numpy as jnp
from jax import lax
from jax.experimental import pallas as pl
from jax.experimental.pallas import tpu as pltpu
from jax.experimental.pallas import tpu_sc as plsc

N_FIELDS = 26
VOCAB = 100000
EMB_DIM = 16
N_CONT = 13
B = 16384
EMB_COLS = N_FIELDS * EMB_DIM
EPS = 1e-5

CH = 128
BLK = 512
H1P = 1024
H2P = 512
CONTP = 16
OUTP = 128


CHB = 8192


def _sc_gather(tableT, xcatT):
    info = plsc.get_sparse_core_info()
    nc, ns = info.num_cores, info.num_subcores
    nw = nc * ns
    npairs = N_FIELDS * EMB_DIM
    per_w = npairs // nw
    assert npairs == per_w * nw
    nchunk = B // CHB
    mesh = plsc.VectorSubcoreMesh(core_axis_name="c", subcore_axis_name="s")

    @functools.partial(
        pl.kernel,
        mesh=mesh,
        out_type=jax.ShapeDtypeStruct((npairs, B), jnp.float32),
        scratch_types=[
            pltpu.VMEM((VOCAB,), jnp.float32),
            pltpu.VMEM((CHB,), jnp.int32),
            pltpu.VMEM((CHB,), jnp.int32),
            pltpu.VMEM((CHB,), jnp.float32),
            pltpu.SemaphoreType.DMA,
            pltpu.SemaphoreType.DMA,
        ],
        compiler_params=pltpu.CompilerParams(needs_layout_passes=False),
    )
    def gk(table_hbm, idx_hbm, out_hbm, row_v, idx_v0, idx_v1, o_v,
           rsem, s0):
        wid = lax.axis_index("s") * nc + lax.axis_index("c")
        idx_bufs = (idx_v0, idx_v1)

        def pair_body(p, carry):
            pid = wid * per_w + p
            f = pid // EMB_DIM
            d = pid % EMB_DIM
            rcp = pltpu.async_copy(table_hbm.at[f, d], row_v, rsem)

            @pl.when((p == 0) | (d == 0))
            def _():
                for cb in range(nchunk):
                    pltpu.sync_copy(idx_hbm.at[f, pl.ds(cb * CHB, CHB)],
                                    idx_bufs[cb])

            rcp.wait()

            for cb in range(nchunk):
                idx_v = idx_bufs[cb]

                @pl.when((p > 0) | (cb > 0))
                def _():
                    pltpu.make_async_copy(
                        o_v, out_hbm.at[pid, pl.ds(cb * CHB, CHB)],
                        s0).wait()

                @plsc.parallel_loop(0, CHB // 256, unroll=1)
                def _(j):
                    base = j * 256
                    idxs = [idx_v[pl.ds(base + u * 16, 16)]
                            for u in range(16)]
                    vals = [plsc.load_gather(row_v, [ix]) for ix in idxs]
                    for u in range(16):
                        o_v[pl.ds(base + u * 16, 16)] = vals[u]

                pltpu.async_copy(
                    o_v, out_hbm.at[pid, pl.ds(cb * CHB, CHB)], s0)
            return carry

        lax.fori_loop(0, per_w, pair_body, 0)
        pltpu.make_async_copy(o_v, out_hbm.at[0, pl.ds(0, CHB)], s0).wait()

    return gk(tableT, xcatT)


NBLK = B // BLK


def _tc_mlp(xembT, xc, w1e, w1c, b1, bncg, bncb, w2, b2, g1, bb1,
            w3, b3, g2, bb2):

    def body(xt_ref, xc_ref, w1e_ref, w1c_ref, b1_ref, bncg_ref, bncb_ref,
             w2_ref, b2_ref, g1_ref, bb1_ref, w3_ref, b3_ref, g2_ref,
             bb2_ref, o_ref, h1_s, h2_s, st1, st2, affc, w2s, c2, w3s, c3):
        p = pl.program_id(0)
        i = pl.program_id(1)

        @pl.when((p == 0) & (i == 0))
        def _():
            xcf = xc_ref[...].astype(jnp.float32)
            m = jnp.mean(xcf, axis=0, keepdims=True)
            v = jnp.mean(xcf * xcf, axis=0, keepdims=True) - m * m
            sc = bncg_ref[...] / jnp.sqrt(v + EPS)
            affc[0:1, :] = sc
            affc[1:2, :] = bncb_ref[...] - m * sc
            st1[...] = jnp.zeros_like(st1)
            st2[...] = jnp.zeros_like(st2)
        ones = jnp.ones((1, BLK), jnp.float32)
        cdot = lambda a, b: lax.dot_general(
            a, b, (((0,), (0,)), ((), ())),
            preferred_element_type=jnp.float32)

        @pl.when(p == 0)
        def _():
            xcb = xc_ref[pl.ds(i * BLK, BLK), :].astype(jnp.float32)
            xcn = xcb * affc[0:1, :] + affc[1:2, :]
            h = lax.dot_general(xt_ref[...], w1e_ref[...],
                                (((0,), (0,)), ((), ())),
                                preferred_element_type=jnp.float32)
            h = h + jnp.dot(xcn, w1c_ref[...],
                            preferred_element_type=jnp.float32)
            h = jnp.maximum(h + b1_ref[...], 0.0)
            h1_s[pl.ds(i * BLK, BLK), :] = h.astype(jnp.bfloat16)
            st1[0:1, :] += jnp.dot(ones, h, preferred_element_type=jnp.float32)
            st1[1:2, :] += jnp.dot(ones, h * h, preferred_element_type=jnp.float32)

        @pl.when((p == 1) & (i == 0))
        def _():
            m = st1[0:1, :] * (1.0 / B)
            v = st1[1:2, :] * (1.0 / B) - m * m
            sc = g1_ref[...] / jnp.sqrt(v + EPS)
            sh = bb1_ref[...] - m * sc
            bT = jnp.swapaxes(jnp.concatenate([sc, sh], axis=0), 0, 1)
            w2f = w2_ref[...].astype(jnp.float32)
            w2s[...] = (w2f * bT[:, 0:1]).astype(jnp.bfloat16)
            c2[0:1, :] = b2_ref[...] + cdot(bT[:, 1:2], w2f)

        @pl.when(p == 1)
        def _():
            h1b = h1_s[pl.ds(i * BLK, BLK), :]
            h = jnp.dot(h1b, w2s[...], preferred_element_type=jnp.float32)
            h = jnp.maximum(h + c2[0:1, :], 0.0)
            h2_s[pl.ds(i * BLK, BLK), :] = h.astype(jnp.bfloat16)
            st2[0:1, :] += jnp.dot(ones, h, preferred_element_type=jnp.float32)
            st2[1:2, :] += jnp.dot(ones, h * h, preferred_element_type=jnp.float32)

        @pl.when((p == 2) & (i == 0))
        def _():
            m = st2[0:1, :] * (1.0 / B)
            v = st2[1:2, :] * (1.0 / B) - m * m
            sc = g2_ref[...] / jnp.sqrt(v + EPS)
            sh = bb2_ref[...] - m * sc
            bT = jnp.swapaxes(jnp.concatenate([sc, sh], axis=0), 0, 1)
            w3f = w3_ref[...].astype(jnp.float32)
            w3s[...] = (w3f * bT[:, 0:1]).astype(jnp.bfloat16)
            c3[0:1, :] = b3_ref[...] + cdot(bT[:, 1:2], w3f)

        @pl.when(p == 2)
        def _():
            h2b = h2_s[pl.ds(i * BLK, BLK), :]
            o = jnp.dot(h2b, w3s[...], preferred_element_type=jnp.float32)
            o_ref[...] = o + c3[0:1, :]

    cnst = lambda p, i: (0, 0)
    return pl.pallas_call(
        body,
        grid=(3, NBLK),
        in_specs=[
            pl.BlockSpec((EMB_COLS, BLK),
                         lambda p, i: (0, jnp.where(p == 0, i, NBLK - 1))),
            pl.BlockSpec((B, CONTP), cnst),
            pl.BlockSpec((EMB_COLS, H1P), cnst),
            pl.BlockSpec((CONTP, H1P), cnst),
            pl.BlockSpec((1, H1P), cnst),
            pl.BlockSpec((1, CONTP), cnst),
            pl.BlockSpec((1, CONTP), cnst),
            pl.BlockSpec((H1P, H2P), cnst),
            pl.BlockSpec((1, H2P), cnst),
            pl.BlockSpec((1, H1P), cnst),
            pl.BlockSpec((1, H1P), cnst),
            pl.BlockSpec((H2P, OUTP), cnst),
            pl.BlockSpec((1, OUTP), cnst),
            pl.BlockSpec((1, H2P), cnst),
            pl.BlockSpec((1, H2P), cnst),
        ],
        out_specs=pl.BlockSpec((BLK, OUTP),
                               lambda p, i: (jnp.where(p == 2, i, 0), 0)),
        out_shape=jax.ShapeDtypeStruct((B, OUTP), jnp.float32),
        scratch_shapes=[
            pltpu.VMEM((B, H1P), jnp.bfloat16),
            pltpu.VMEM((B, H2P), jnp.bfloat16),
            pltpu.VMEM((8, H1P), jnp.float32),
            pltpu.VMEM((8, H2P), jnp.float32),
            pltpu.VMEM((8, CONTP), jnp.float32),
            pltpu.VMEM((H1P, H2P), jnp.bfloat16),
            pltpu.VMEM((8, H2P), jnp.float32),
            pltpu.VMEM((H2P, OUTP), jnp.bfloat16),
            pltpu.VMEM((8, OUTP), jnp.float32),
        ],
        compiler_params=pltpu.CompilerParams(
            dimension_semantics=("arbitrary", "arbitrary"),
            vmem_limit_bytes=64 * 1024 * 1024),
    )(xembT, xc, w1e, w1c, b1, bncg, bncb, w2, b2, g1, bb1, w3, b3, g2, bb2)


def kernel(x_cat, x_cont, emb_tables, bn_cont_g, bn_cont_b,
           W1, b1, bn1_g, bn1_b, W2, b2, bn2_g, bn2_b, W3, b3):
    tableT = jnp.transpose(emb_tables, (0, 2, 1))
    xcatT = jnp.transpose(x_cat.astype(jnp.int32), (1, 0))

    xc = jnp.pad(x_cont, ((0, 0), (0, CONTP - N_CONT))).astype(jnp.bfloat16)
    bncg = jnp.pad(bn_cont_g, (0, CONTP - N_CONT)).reshape(1, CONTP)
    bncb = jnp.pad(bn_cont_b, (0, CONTP - N_CONT)).reshape(1, CONTP)

    n1 = W1.shape[0]
    n2 = W2.shape[0]
    w1e = jnp.pad(W1[:, :EMB_COLS].T, ((0, 0), (0, H1P - n1)))
    w1c = jnp.pad(W1[:, EMB_COLS:].T,
                  ((0, CONTP - N_CONT), (0, H1P - n1)))
    b1p = jnp.pad(b1, (0, H1P - n1)).reshape(1, H1P)
    g1p = jnp.pad(bn1_g, (0, H1P - n1)).reshape(1, H1P)
    bb1p = jnp.pad(bn1_b, (0, H1P - n1)).reshape(1, H1P)

    w2p = jnp.pad(W2.T, ((0, H1P - n1), (0, H2P - n2))).astype(jnp.bfloat16)
    b2p = jnp.pad(b2, (0, H2P - n2)).reshape(1, H2P)
    g2p = jnp.pad(bn2_g, (0, H2P - n2)).reshape(1, H2P)
    bb2p = jnp.pad(bn2_b, (0, H2P - n2)).reshape(1, H2P)

    w3p = jnp.pad(W3.T, ((0, H2P - n2), (0, OUTP - 1))).astype(jnp.bfloat16)
    b3p = jnp.pad(b3, (0, OUTP - 1)).reshape(1, OUTP)

    xembT = _sc_gather(tableT, xcatT)
    o = _tc_mlp(xembT, xc, w1e, w1c, b1p, bncg, bncb,
                w2p, b2p, g1p, bb1p, w3p, b3p, g2p, bb2p)
    return o[:, :1]


# speedup vs baseline: 1.2371x; 1.0005x over previous
import functools

import jax
import jax.numpy as jnp
from jax import lax
from jax.experimental import pallas as pl
from jax.experimental.pallas import tpu as pltpu
from jax.experimental.pallas import tpu_sc as plsc

N_FIELDS = 26
VOCAB = 100000
EMB_DIM = 16
N_CONT = 13
B = 16384
EMB_COLS = N_FIELDS * EMB_DIM
EPS = 1e-5

BLK = 512
H1P = 1024
H2P = 512
CONTP = 16
OUTP = 128


CHB = 8192


def _sc_gather(tableT, xcatT):
    info = plsc.get_sparse_core_info()
    nc, ns = info.num_cores, info.num_subcores
    nw = nc * ns
    npairs = N_FIELDS * EMB_DIM
    per_w = npairs // nw
    assert npairs == per_w * nw
    nchunk = B // CHB
    mesh = plsc.VectorSubcoreMesh(core_axis_name="c", subcore_axis_name="s")

    @functools.partial(
        pl.kernel,
        mesh=mesh,
        out_type=jax.ShapeDtypeStruct((npairs, B), jnp.float32),
        scratch_types=[
            pltpu.VMEM((VOCAB,), jnp.float32),
            pltpu.VMEM((CHB,), jnp.int32),
            pltpu.VMEM((CHB,), jnp.int32),
            pltpu.VMEM((CHB,), jnp.float32),
            pltpu.SemaphoreType.DMA,
            pltpu.SemaphoreType.DMA,
        ],
        compiler_params=pltpu.CompilerParams(needs_layout_passes=False),
    )
    def gk(table_hbm, idx_hbm, out_hbm, row_v, idx_v0, idx_v1, o_v,
           rsem, s0):
        wid = lax.axis_index("s") * nc + lax.axis_index("c")
        idx_bufs = (idx_v0, idx_v1)

        def pair_body(p, carry):
            pid = wid * per_w + p
            f = pid // EMB_DIM
            d = pid % EMB_DIM
            rcp = pltpu.async_copy(table_hbm.at[f, d], row_v, rsem)

            @pl.when((p == 0) | (d == 0))
            def _():
                for cb in range(nchunk):
                    pltpu.sync_copy(idx_hbm.at[f, pl.ds(cb * CHB, CHB)],
                                    idx_bufs[cb])

            rcp.wait()

            for cb in range(nchunk):
                idx_v = idx_bufs[cb]

                @pl.when((p > 0) | (cb > 0))
                def _():
                    pltpu.make_async_copy(
                        o_v, out_hbm.at[pid, pl.ds(cb * CHB, CHB)],
                        s0).wait()

                @plsc.parallel_loop(0, CHB // 256, unroll=1)
                def _(j):
                    base = j * 256
                    idxs = [idx_v[pl.ds(base + u * 16, 16)]
                            for u in range(16)]
                    vals = [plsc.load_gather(row_v, [ix]) for ix in idxs]
                    for u in range(16):
                        o_v[pl.ds(base + u * 16, 16)] = vals[u]

                pltpu.async_copy(
                    o_v, out_hbm.at[pid, pl.ds(cb * CHB, CHB)], s0)
            return carry

        lax.fori_loop(0, per_w, pair_body, 0)
        pltpu.make_async_copy(o_v, out_hbm.at[0, pl.ds(0, CHB)], s0).wait()

    return gk(tableT, xcatT)


NBLK = B // BLK


def _tc_mlp(xembT, xc, w1e, w1c, b1, bncg, bncb, w2, b2, g1, bb1,
            w3, b3, g2, bb2):

    def body(xt_ref, xc_ref, w1e_ref, w1c_ref, b1_ref, bncg_ref, bncb_ref,
             w2_ref, b2_ref, g1_ref, bb1_ref, w3_ref, b3_ref, g2_ref,
             bb2_ref, o_ref, h1_s, h2_s, st1, st2, affc, w2s, c2, w3s, c3):
        p = pl.program_id(0)
        i = pl.program_id(1)

        @pl.when((p == 0) & (i == 0))
        def _():
            xcf = xc_ref[...].astype(jnp.float32)
            m = jnp.mean(xcf, axis=0, keepdims=True)
            v = jnp.mean(xcf * xcf, axis=0, keepdims=True) - m * m
            sc = bncg_ref[...] / jnp.sqrt(v + EPS)
            affc[0:1, :] = sc
            affc[1:2, :] = bncb_ref[...] - m * sc
            st1[...] = jnp.zeros_like(st1)
            st2[...] = jnp.zeros_like(st2)
        ones = jnp.ones((1, BLK), jnp.float32)
        cdot = lambda a, b: lax.dot_general(
            a, b, (((0,), (0,)), ((), ())),
            preferred_element_type=jnp.float32)

        @pl.when(p == 0)
        def _():
            xcb = xc_ref[pl.ds(i * BLK, BLK), :].astype(jnp.float32)
            xcn = xcb * affc[0:1, :] + affc[1:2, :]
            h = lax.dot_general(xt_ref[...], w1e_ref[...],
                                (((0,), (0,)), ((), ())),
                                preferred_element_type=jnp.float32)
            h = h + jnp.dot(xcn, w1c_ref[...],
                            preferred_element_type=jnp.float32)
            h = jnp.maximum(h + b1_ref[...], 0.0)
            h1_s[pl.ds(i * BLK, BLK), :] = h.astype(jnp.bfloat16)
            st1[0:1, :] += jnp.dot(ones, h, preferred_element_type=jnp.float32)
            st1[1:2, :] += jnp.dot(ones, h * h, preferred_element_type=jnp.float32)

        @pl.when((p == 1) & (i == 0))
        def _():
            m = st1[0:1, :] * (1.0 / B)
            v = st1[1:2, :] * (1.0 / B) - m * m
            sc = g1_ref[...] / jnp.sqrt(v + EPS)
            sh = bb1_ref[...] - m * sc
            bT = jnp.swapaxes(jnp.concatenate([sc, sh], axis=0), 0, 1)
            w2f = w2_ref[...].astype(jnp.float32)
            w2s[...] = (w2f * bT[:, 0:1]).astype(jnp.bfloat16)
            c2[0:1, :] = b2_ref[...] + cdot(bT[:, 1:2], w2f)

        @pl.when(p == 1)
        def _():
            h1b = h1_s[pl.ds(i * BLK, BLK), :]
            h = jnp.dot(h1b, w2s[...], preferred_element_type=jnp.float32)
            h = jnp.maximum(h + c2[0:1, :], 0.0)
            h2_s[pl.ds(i * BLK, BLK), :] = h.astype(jnp.bfloat16)
            st2[0:1, :] += jnp.dot(ones, h, preferred_element_type=jnp.float32)
            st2[1:2, :] += jnp.dot(ones, h * h, preferred_element_type=jnp.float32)

        @pl.when((p == 2) & (i == 0))
        def _():
            m = st2[0:1, :] * (1.0 / B)
            v = st2[1:2, :] * (1.0 / B) - m * m
            sc = g2_ref[...] / jnp.sqrt(v + EPS)
            sh = bb2_ref[...] - m * sc
            bT = jnp.swapaxes(jnp.concatenate([sc, sh], axis=0), 0, 1)
            w3f = w3_ref[...].astype(jnp.float32)
            w3s[...] = (w3f * bT[:, 0:1]).astype(jnp.bfloat16)
            c3[0:1, :] = b3_ref[...] + cdot(bT[:, 1:2], w3f)

        @pl.when(p == 2)
        def _():
            h2b = h2_s[pl.ds(i * BLK, BLK), :]
            o = jnp.dot(h2b, w3s[...], preferred_element_type=jnp.float32)
            o_ref[...] = o + c3[0:1, :]

    cnst = lambda p, i: (0, 0)
    return pl.pallas_call(
        body,
        grid=(3, NBLK),
        in_specs=[
            pl.BlockSpec((EMB_COLS, BLK),
                         lambda p, i: (0, jnp.where(p == 0, i, NBLK - 1))),
            pl.BlockSpec((B, CONTP), cnst),
            pl.BlockSpec((EMB_COLS, H1P), cnst),
            pl.BlockSpec((CONTP, H1P), cnst),
            pl.BlockSpec((1, H1P), cnst),
            pl.BlockSpec((1, CONTP), cnst),
            pl.BlockSpec((1, CONTP), cnst),
            pl.BlockSpec((H1P, H2P), cnst),
            pl.BlockSpec((1, H2P), cnst),
            pl.BlockSpec((1, H1P), cnst),
            pl.BlockSpec((1, H1P), cnst),
            pl.BlockSpec((H2P, OUTP), cnst),
            pl.BlockSpec((1, OUTP), cnst),
            pl.BlockSpec((1, H2P), cnst),
            pl.BlockSpec((1, H2P), cnst),
        ],
        out_specs=pl.BlockSpec((BLK, OUTP),
                               lambda p, i: (jnp.where(p == 2, i, 0), 0)),
        out_shape=jax.ShapeDtypeStruct((B, OUTP), jnp.float32),
        scratch_shapes=[
            pltpu.VMEM((B, H1P), jnp.bfloat16),
            pltpu.VMEM((B, H2P), jnp.bfloat16),
            pltpu.VMEM((8, H1P), jnp.float32),
            pltpu.VMEM((8, H2P), jnp.float32),
            pltpu.VMEM((8, CONTP), jnp.float32),
            pltpu.VMEM((H1P, H2P), jnp.bfloat16),
            pltpu.VMEM((8, H2P), jnp.float32),
            pltpu.VMEM((H2P, OUTP), jnp.bfloat16),
            pltpu.VMEM((8, OUTP), jnp.float32),
        ],
        compiler_params=pltpu.CompilerParams(
            dimension_semantics=("arbitrary", "arbitrary"),
            vmem_limit_bytes=64 * 1024 * 1024),
    )(xembT, xc, w1e, w1c, b1, bncg, bncb, w2, b2, g1, bb1, w3, b3, g2, bb2)


def kernel(x_cat, x_cont, emb_tables, bn_cont_g, bn_cont_b,
           W1, b1, bn1_g, bn1_b, W2, b2, bn2_g, bn2_b, W3, b3):
    tableT = jnp.transpose(emb_tables, (0, 2, 1))
    xcatT = jnp.transpose(x_cat.astype(jnp.int32), (1, 0))

    xc = jnp.pad(x_cont, ((0, 0), (0, CONTP - N_CONT))).astype(jnp.bfloat16)
    bncg = jnp.pad(bn_cont_g, (0, CONTP - N_CONT)).reshape(1, CONTP)
    bncb = jnp.pad(bn_cont_b, (0, CONTP - N_CONT)).reshape(1, CONTP)

    n1 = W1.shape[0]
    n2 = W2.shape[0]
    w1e = jnp.pad(W1[:, :EMB_COLS].T, ((0, 0), (0, H1P - n1)))
    w1c = jnp.pad(W1[:, EMB_COLS:].T,
                  ((0, CONTP - N_CONT), (0, H1P - n1)))
    b1p = jnp.pad(b1, (0, H1P - n1)).reshape(1, H1P)
    g1p = jnp.pad(bn1_g, (0, H1P - n1)).reshape(1, H1P)
    bb1p = jnp.pad(bn1_b, (0, H1P - n1)).reshape(1, H1P)

    w2p = jnp.pad(W2.T, ((0, H1P - n1), (0, H2P - n2))).astype(jnp.bfloat16)
    b2p = jnp.pad(b2, (0, H2P - n2)).reshape(1, H2P)
    g2p = jnp.pad(bn2_g, (0, H2P - n2)).reshape(1, H2P)
    bb2p = jnp.pad(bn2_b, (0, H2P - n2)).reshape(1, H2P)

    w3p = jnp.pad(W3.T, ((0, H2P - n2), (0, OUTP - 1))).astype(jnp.bfloat16)
    b3p = jnp.pad(b3, (0, OUTP - 1)).reshape(1, OUTP)

    xembT = _sc_gather(tableT, xcatT)
    o = _tc_mlp(xembT, xc, w1e, w1c, b1p, bncg, bncb,
                w2p, b2p, g1p, bb1p, w3p, b3p, g2p, bb2p)
    return o[:, :1]
